# baseline (speedup 1.0000x reference)
import functools

import jax
import jax.numpy as jnp
from jax import lax
from jax.experimental import pallas as pl
from jax.experimental.pallas import tpu as pltpu
from jax.experimental.pallas import tpu_sc as plsc

N = 10000
E = 160000
D = 256
H = 128
L = 5
NT = 120
NC = 4
EPS = 1e-5

N_PAD = 10240
BN_ROWS = 512
N_BLOCKS = N_PAD // BN_ROWS
ROWS_PER_TILE = N_PAD // 16

CHUNK = 128
ROWS_CH = CHUNK
PHASES = 2
PH_CHUNKS = 40
E_TILE = PHASES * PH_CHUNKS * CHUNK
E_PAD = E_TILE * 16

DEG_TILE = 5120
DEG_CHUNKS = DEG_TILE // CHUNK
DEG_PAD = DEG_TILE * 32

PAD_ROW = 10000

def _deg_body(dst_hbm, deg_out, didx, ones_v, zbuf, deg_sh):
    c = lax.axis_index("c")
    s = lax.axis_index("s")
    pltpu.sync_copy(dst_hbm.at[c, s], didx)

    z16 = jnp.zeros((16,), jnp.float32)
    o16 = jnp.ones((16,), jnp.float32)

    def fill_ones(j, carry):
        ones_v[pl.ds(j * 16, 16)] = o16
        return carry

    lax.fori_loop(0, CHUNK // 16, fill_ones, 0)

    def fill_zero(j, carry):
        zbuf[pl.ds(j * 16, 16)] = z16
        return carry

    lax.fori_loop(0, ROWS_PER_TILE // 16, fill_zero, 0)

    pltpu.sync_copy(zbuf, deg_sh.at[pl.ds(s * ROWS_PER_TILE, ROWS_PER_TILE)])
    plsc.subcore_barrier()

    def body(k, carry):
        pltpu.sync_copy(ones_v, deg_sh.at[didx.at[k]], add=True)
        return carry

    lax.fori_loop(0, DEG_CHUNKS, body, 0)
    plsc.subcore_barrier()

    pltpu.sync_copy(
        deg_sh.at[pl.ds(s * ROWS_PER_TILE, ROWS_PER_TILE)],
        deg_out.at[c, pl.ds(s * ROWS_PER_TILE, ROWS_PER_TILE)],
    )


def _scatter_body(scaled_hbm, sidx_hbm, didx_hbm, zeros_hbm, agg_out,
                  sidx, didx, g0, agg_sh, gs0):
    c = lax.axis_index("c")
    s = lax.axis_index("s")
    base = s * ROWS_PER_TILE
    pltpu.sync_copy(zeros_hbm, agg_sh.at[pl.ds(base, ROWS_PER_TILE)])
    plsc.subcore_barrier()

    for h in range(PHASES):
        pltpu.sync_copy(sidx_hbm.at[c, s, h], sidx)
        pltpu.sync_copy(didx_hbm.at[s, h], didx)

        def body(k, carry):
            pltpu.async_copy(scaled_hbm.at[sidx.at[k]], g0, gs0).wait()
            pltpu.sync_copy(g0, agg_sh.at[didx.at[k]], add=True)
            return carry

        lax.fori_loop(0, PH_CHUNKS, body, 0)

    plsc.subcore_barrier()

    pltpu.sync_copy(
        agg_sh.at[pl.ds(base, ROWS_PER_TILE)],
        agg_out.at[c, pl.ds(base, ROWS_PER_TILE)],
    )


@functools.lru_cache(maxsize=1)
def _sc_kernels():
    mesh = plsc.VectorSubcoreMesh(core_axis_name="c", subcore_axis_name="s")
    deg_kernel = pl.kernel(
        _deg_body,
        out_type=jax.ShapeDtypeStruct((2, N_PAD), jnp.float32),
        mesh=mesh,
        scratch_types=[
            pltpu.VMEM((DEG_CHUNKS, CHUNK), jnp.int32),
            pltpu.VMEM((CHUNK,), jnp.float32),
            pltpu.VMEM((ROWS_PER_TILE,), jnp.float32),
            pltpu.VMEM_SHARED((N_PAD,), jnp.float32),
        ],
    )
    scatter_kernel = pl.kernel(
        _scatter_body,
        out_type=jax.ShapeDtypeStruct((2, N_PAD, H), jnp.float32),
        mesh=mesh,
        scratch_types=[
            pltpu.VMEM((PH_CHUNKS, CHUNK), jnp.int32),
            pltpu.VMEM((PH_CHUNKS, CHUNK), jnp.int32),
            pltpu.VMEM((ROWS_CH, H), jnp.float32),
            pltpu.VMEM_SHARED((N_PAD, H), jnp.float32),
            pltpu.SemaphoreType.DMA,
        ],
    )
    return deg_kernel, scatter_kernel


def _l1_body(nt_ref, ch_ref, T_ref, deg_ref, W_ref, b_ref, out_ref, dinv_ref):
    nt = nt_ref[...]
    ch = ch_ref[...]
    ids = lax.broadcasted_iota(jnp.int32, (BN_ROWS, D), 1)
    m = (ids == nt).astype(jnp.float32) + (ids == (ch + NT)).astype(jnp.float32)
    x = jnp.dot(m, T_ref[...], preferred_element_type=jnp.float32)
    deg = deg_ref[0, :] + deg_ref[1, :] + 1.0
    dinv = lax.rsqrt(deg)[:, None]
    y = (jnp.dot(x, W_ref[...], preferred_element_type=jnp.float32)
         + b_ref[...]) * dinv
    out_ref[0] = y[:, :H]
    out_ref[1] = y[:, H:]
    dinv_ref[...] = dinv


def _li_body(agg_ref, scp_ref, dinv_ref, g_ref, be_ref, mu_ref, va_ref,
             W_ref, b_ref, out_ref):
    dinv = dinv_ref[...]
    hpre = jnp.concatenate(
        [agg_ref[0] + scp_ref[0], agg_ref[1] + scp_ref[1]], axis=1) * dinv
    sbn = g_ref[...] * lax.rsqrt(va_ref[...] + EPS)
    h = sbn * (hpre - mu_ref[...]) + be_ref[...]
    h = jnp.maximum(h, 0.0)
    y = (jnp.dot(h, W_ref[...], preferred_element_type=jnp.float32)
         + b_ref[...]) * dinv
    out_ref[0] = y[:, :H]
    out_ref[1] = y[:, H:]


def _ep_body(agg_ref, scp_ref, dinv_ref, g_ref, be_ref, mu_ref, va_ref,
             out_ref):
    dinv = dinv_ref[...]
    hpre = jnp.concatenate(
        [agg_ref[0] + scp_ref[0], agg_ref[1] + scp_ref[1]], axis=1) * dinv
    sbn = g_ref[...] * lax.rsqrt(va_ref[...] + EPS)
    out_ref[...] = sbn * (hpre - mu_ref[...]) + be_ref[...]


def _col_spec(i):
    return (i, 0)


_spec_rows1 = pl.BlockSpec((BN_ROWS, 1), lambda i: (i, 0))
_spec_deg = pl.BlockSpec((2, BN_ROWS), lambda i: (0, i))
_spec_mat = pl.BlockSpec((D, D), lambda i: (0, 0))
_spec_vec = pl.BlockSpec((1, D), lambda i: (0, 0))
_spec_split = pl.BlockSpec((2, BN_ROWS, H), lambda i: (0, i, 0))
_spec_full = pl.BlockSpec((BN_ROWS, D), lambda i: (i, 0))

_l1_call = pl.pallas_call(
    _l1_body,
    grid=(N_BLOCKS,),
    in_specs=[_spec_rows1, _spec_rows1, _spec_mat, _spec_deg, _spec_mat,
              _spec_vec],
    out_specs=[_spec_split, _spec_rows1],
    out_shape=[
        jax.ShapeDtypeStruct((2, N_PAD, H), jnp.float32),
        jax.ShapeDtypeStruct((N_PAD, 1), jnp.float32),
    ],
)

_li_call = pl.pallas_call(
    _li_body,
    grid=(N_BLOCKS,),
    in_specs=[_spec_split, _spec_split, _spec_rows1, _spec_vec, _spec_vec,
              _spec_vec, _spec_vec, _spec_mat, _spec_vec],
    out_specs=_spec_split,
    out_shape=jax.ShapeDtypeStruct((2, N_PAD, H), jnp.float32),
)

_ep_call = pl.pallas_call(
    _ep_body,
    grid=(N_BLOCKS,),
    in_specs=[_spec_split, _spec_split, _spec_rows1, _spec_vec, _spec_vec,
              _spec_vec, _spec_vec],
    out_specs=_spec_full,
    out_shape=jax.ShapeDtypeStruct((N_PAD, D), jnp.float32),
)


@jax.jit
def kernel(node_types, chirality, edge_index, node_emb, chir_emb, W, b,
           gamma, beta, run_mean, run_var):
    nt = jnp.zeros((N_PAD, 1), jnp.int32).at[:N, 0].set(
        node_types.astype(jnp.int32))
    ch = jnp.zeros((N_PAD, 1), jnp.int32).at[:N, 0].set(
        chirality.astype(jnp.int32))
    src = edge_index[0].astype(jnp.int32)
    dst = edge_index[1].astype(jnp.int32)

    T = jnp.zeros((D, D), jnp.float32)
    T = T.at[:NT].set(node_emb).at[NT:NT + NC].set(chir_emb)

    dst_deg = jnp.full((DEG_PAD,), PAD_ROW, jnp.int32).at[:E].set(dst)
    dst_deg = dst_deg.reshape(2, 16, DEG_CHUNKS, CHUNK)

    srcp = jnp.full((E_PAD,), PAD_ROW, jnp.int32).at[:E].set(src)
    srcp = srcp.reshape(16, PHASES, PH_CHUNKS, CHUNK)
    sidx = jnp.stack([srcp, srcp + N_PAD])
    didx = jnp.full((E_PAD,), PAD_ROW, jnp.int32).at[:E].set(dst)
    didx = didx.reshape(16, PHASES, PH_CHUNKS, CHUNK)

    zeros_blk = jnp.zeros((ROWS_PER_TILE, H), jnp.float32)

    deg_kernel, scatter_kernel = _sc_kernels()
    deg2 = deg_kernel(dst_deg)

    scaled, dinv = _l1_call(nt, ch, T, deg2, W[0], b[0][None, :])
    for i in range(1, L + 1):
        agg = scatter_kernel(scaled.reshape(2 * N_PAD, H), sidx, didx,
                             zeros_blk)
        if i < L:
            scaled = _li_call(agg, scaled, dinv,
                              gamma[i - 1][None, :], beta[i - 1][None, :],
                              run_mean[i - 1][None, :], run_var[i - 1][None, :],
                              W[i], b[i][None, :])
        else:
            out = _ep_call(agg, scaled, dinv,
                           gamma[L - 1][None, :], beta[L - 1][None, :],
                           run_mean[L - 1][None, :], run_var[L - 1][None, :])
    return out[:N]


# speedup vs baseline: 1.0017x; 1.0017x over previous
import functools

import jax
import jax.numpy as jnp
from jax import lax
from jax.experimental import pallas as pl
from jax.experimental.pallas import tpu as pltpu
from jax.experimental.pallas import tpu_sc as plsc

N = 10000
E = 160000
D = 256
H = 128
L = 5
NT = 120
NC = 4
EPS = 1e-5

N_PAD = 10240
BN_ROWS = 512
N_BLOCKS = N_PAD // BN_ROWS
ROWS_PER_TILE = N_PAD // 16

CHUNK = 128
N_CHUNKS = 80
E_TILE = N_CHUNKS * CHUNK
E_PAD = E_TILE * 16

DEG_TILE = 5120
DEG_CHUNKS = DEG_TILE // CHUNK
DEG_PAD = DEG_TILE * 32

PAD_ROW = 10000

def _deg_body(dst_hbm, deg_out, didx, ones_v, zbuf, deg_sh):
    c = lax.axis_index("c")
    s = lax.axis_index("s")
    pltpu.sync_copy(dst_hbm.at[c, s], didx)

    z16 = jnp.zeros((16,), jnp.float32)
    o16 = jnp.ones((16,), jnp.float32)

    def fill_ones(j, carry):
        ones_v[pl.ds(j * 16, 16)] = o16
        return carry

    lax.fori_loop(0, CHUNK // 16, fill_ones, 0)

    def fill_zero(j, carry):
        zbuf[pl.ds(j * 16, 16)] = z16
        return carry

    lax.fori_loop(0, ROWS_PER_TILE // 16, fill_zero, 0)

    pltpu.sync_copy(zbuf, deg_sh.at[pl.ds(s * ROWS_PER_TILE, ROWS_PER_TILE)])
    plsc.subcore_barrier()

    def body(k, carry):
        pltpu.sync_copy(ones_v, deg_sh.at[didx.at[k]], add=True)
        return carry

    lax.fori_loop(0, DEG_CHUNKS, body, 0)
    plsc.subcore_barrier()

    pltpu.sync_copy(
        deg_sh.at[pl.ds(s * ROWS_PER_TILE, ROWS_PER_TILE)],
        deg_out.at[c, pl.ds(s * ROWS_PER_TILE, ROWS_PER_TILE)],
    )


def _scatter_body(scaled_hbm, sidx_hbm, didx_hbm, zeros_hbm, agg_out,
                  sidx, didx, g0, agg_sh, gs0):
    c = lax.axis_index("c")
    s = lax.axis_index("s")
    base = s * ROWS_PER_TILE
    pltpu.sync_copy(zeros_hbm, agg_sh.at[pl.ds(base, ROWS_PER_TILE)])
    plsc.subcore_barrier()

    pltpu.sync_copy(sidx_hbm.at[c, s], sidx)
    pltpu.sync_copy(didx_hbm.at[s], didx)

    def body(k, carry):
        pltpu.async_copy(scaled_hbm.at[sidx.at[k]], g0, gs0).wait()
        pltpu.sync_copy(g0, agg_sh.at[didx.at[k]], add=True)
        return carry

    lax.fori_loop(0, N_CHUNKS, body, 0)

    plsc.subcore_barrier()

    pltpu.sync_copy(
        agg_sh.at[pl.ds(base, ROWS_PER_TILE)],
        agg_out.at[c, pl.ds(base, ROWS_PER_TILE)],
    )


@functools.lru_cache(maxsize=1)
def _sc_kernels():
    mesh = plsc.VectorSubcoreMesh(core_axis_name="c", subcore_axis_name="s")
    deg_kernel = pl.kernel(
        _deg_body,
        out_type=jax.ShapeDtypeStruct((2, N_PAD), jnp.float32),
        mesh=mesh,
        scratch_types=[
            pltpu.VMEM((DEG_CHUNKS, CHUNK), jnp.int32),
            pltpu.VMEM((CHUNK,), jnp.float32),
            pltpu.VMEM((ROWS_PER_TILE,), jnp.float32),
            pltpu.VMEM_SHARED((N_PAD,), jnp.float32),
        ],
    )
    scatter_kernel = pl.kernel(
        _scatter_body,
        out_type=jax.ShapeDtypeStruct((2, N_PAD, H), jnp.float32),
        mesh=mesh,
        scratch_types=[
            pltpu.VMEM((N_CHUNKS, CHUNK), jnp.int32),
            pltpu.VMEM((N_CHUNKS, CHUNK), jnp.int32),
            pltpu.VMEM((CHUNK, H), jnp.float32),
            pltpu.VMEM_SHARED((N_PAD, H), jnp.float32),
            pltpu.SemaphoreType.DMA,
        ],
    )
    return deg_kernel, scatter_kernel


def _l1_body(nt_ref, ch_ref, T_ref, deg_ref, W_ref, b_ref, out_ref, dinv_ref):
    nt = nt_ref[...]
    ch = ch_ref[...]
    ids = lax.broadcasted_iota(jnp.int32, (BN_ROWS, D), 1)
    m = (ids == nt).astype(jnp.float32) + (ids == (ch + NT)).astype(jnp.float32)
    x = jnp.dot(m, T_ref[...], preferred_element_type=jnp.float32)
    deg = deg_ref[0, :] + deg_ref[1, :] + 1.0
    dinv = lax.rsqrt(deg)[:, None]
    y = (jnp.dot(x, W_ref[...], preferred_element_type=jnp.float32)
         + b_ref[...]) * dinv
    out_ref[0] = y[:, :H]
    out_ref[1] = y[:, H:]
    dinv_ref[...] = dinv


def _li_body(agg_ref, scp_ref, dinv_ref, g_ref, be_ref, mu_ref, va_ref,
             W_ref, b_ref, out_ref):
    dinv = dinv_ref[...]
    hpre = jnp.concatenate(
        [agg_ref[0] + scp_ref[0], agg_ref[1] + scp_ref[1]], axis=1) * dinv
    sbn = g_ref[...] * lax.rsqrt(va_ref[...] + EPS)
    h = sbn * (hpre - mu_ref[...]) + be_ref[...]
    h = jnp.maximum(h, 0.0)
    y = (jnp.dot(h, W_ref[...], preferred_element_type=jnp.float32)
         + b_ref[...]) * dinv
    out_ref[0] = y[:, :H]
    out_ref[1] = y[:, H:]


def _ep_body(agg_ref, scp_ref, dinv_ref, g_ref, be_ref, mu_ref, va_ref,
             out_ref):
    dinv = dinv_ref[...]
    hpre = jnp.concatenate(
        [agg_ref[0] + scp_ref[0], agg_ref[1] + scp_ref[1]], axis=1) * dinv
    sbn = g_ref[...] * lax.rsqrt(va_ref[...] + EPS)
    out_ref[...] = sbn * (hpre - mu_ref[...]) + be_ref[...]


def _col_spec(i):
    return (i, 0)


_spec_rows1 = pl.BlockSpec((BN_ROWS, 1), lambda i: (i, 0))
_spec_deg = pl.BlockSpec((2, BN_ROWS), lambda i: (0, i))
_spec_mat = pl.BlockSpec((D, D), lambda i: (0, 0))
_spec_vec = pl.BlockSpec((1, D), lambda i: (0, 0))
_spec_split = pl.BlockSpec((2, BN_ROWS, H), lambda i: (0, i, 0))
_spec_full = pl.BlockSpec((BN_ROWS, D), lambda i: (i, 0))

_l1_call = pl.pallas_call(
    _l1_body,
    grid=(N_BLOCKS,),
    in_specs=[_spec_rows1, _spec_rows1, _spec_mat, _spec_deg, _spec_mat,
              _spec_vec],
    out_specs=[_spec_split, _spec_rows1],
    out_shape=[
        jax.ShapeDtypeStruct((2, N_PAD, H), jnp.float32),
        jax.ShapeDtypeStruct((N_PAD, 1), jnp.float32),
    ],
)

_li_call = pl.pallas_call(
    _li_body,
    grid=(N_BLOCKS,),
    in_specs=[_spec_split, _spec_split, _spec_rows1, _spec_vec, _spec_vec,
              _spec_vec, _spec_vec, _spec_mat, _spec_vec],
    out_specs=_spec_split,
    out_shape=jax.ShapeDtypeStruct((2, N_PAD, H), jnp.float32),
)

_ep_call = pl.pallas_call(
    _ep_body,
    grid=(N_BLOCKS,),
    in_specs=[_spec_split, _spec_split, _spec_rows1, _spec_vec, _spec_vec,
              _spec_vec, _spec_vec],
    out_specs=_spec_full,
    out_shape=jax.ShapeDtypeStruct((N_PAD, D), jnp.float32),
)


@jax.jit
def kernel(node_types, chirality, edge_index, node_emb, chir_emb, W, b,
           gamma, beta, run_mean, run_var):
    nt = jnp.zeros((N_PAD, 1), jnp.int32).at[:N, 0].set(
        node_types.astype(jnp.int32))
    ch = jnp.zeros((N_PAD, 1), jnp.int32).at[:N, 0].set(
        chirality.astype(jnp.int32))
    src = edge_index[0].astype(jnp.int32)
    dst = edge_index[1].astype(jnp.int32)

    T = jnp.zeros((D, D), jnp.float32)
    T = T.at[:NT].set(node_emb).at[NT:NT + NC].set(chir_emb)

    dst_deg = jnp.full((DEG_PAD,), PAD_ROW, jnp.int32).at[:E].set(dst)
    dst_deg = dst_deg.reshape(2, 16, DEG_CHUNKS, CHUNK)

    srcp = jnp.full((E_PAD,), PAD_ROW, jnp.int32).at[:E].set(src)
    srcp = srcp.reshape(16, N_CHUNKS, CHUNK)
    sidx = jnp.stack([srcp, srcp + N_PAD])
    didx = jnp.full((E_PAD,), PAD_ROW, jnp.int32).at[:E].set(dst)
    didx = didx.reshape(16, N_CHUNKS, CHUNK)

    zeros_blk = jnp.zeros((ROWS_PER_TILE, H), jnp.float32)

    deg_kernel, scatter_kernel = _sc_kernels()
    deg2 = deg_kernel(dst_deg)

    scaled, dinv = _l1_call(nt, ch, T, deg2, W[0], b[0][None, :])
    for i in range(1, L + 1):
        agg = scatter_kernel(scaled.reshape(2 * N_PAD, H), sidx, didx,
                             zeros_blk)
        if i < L:
            scaled = _li_call(agg, scaled, dinv,
                              gamma[i - 1][None, :], beta[i - 1][None, :],
                              run_mean[i - 1][None, :], run_var[i - 1][None, :],
                              W[i], b[i][None, :])
        else:
            out = _ep_call(agg, scaled, dinv,
                           gamma[L - 1][None, :], beta[L - 1][None, :],
                           run_mean[L - 1][None, :], run_var[L - 1][None, :])
    return out[:N]


# speedup vs baseline: 1.7618x; 1.7588x over previous
import functools

import jax
import jax.numpy as jnp
from jax import lax
from jax.experimental import pallas as pl
from jax.experimental.pallas import tpu as pltpu
from jax.experimental.pallas import tpu_sc as plsc

N = 10000
E = 160000
D = 256
H = 128
L = 5
NT = 120
NC = 4
EPS = 1e-5

N_PAD = 10240
BN_ROWS = 512
N_BLOCKS = N_PAD // BN_ROWS
ROWS_PER_TILE = N_PAD // 16

CHUNK = 125
N_CHUNKS = 80
E_TILE = N_CHUNKS * CHUNK

DEG_CHUNKS = 40

def _deg_body(dst_hbm, deg_out, didx, ones_v, zbuf, deg_sh):
    c = lax.axis_index("c")
    s = lax.axis_index("s")
    pltpu.sync_copy(dst_hbm.at[c, s], didx)

    z16 = jnp.zeros((16,), jnp.float32)
    o16 = jnp.ones((16,), jnp.float32)

    def fill_ones(j, carry):
        ones_v[pl.ds(j * 16, 16)] = o16
        return carry

    lax.fori_loop(0, 8, fill_ones, 0)

    def fill_zero(j, carry):
        zbuf[pl.ds(j * 16, 16)] = z16
        return carry

    lax.fori_loop(0, ROWS_PER_TILE // 16, fill_zero, 0)

    pltpu.sync_copy(zbuf, deg_sh.at[pl.ds(s * ROWS_PER_TILE, ROWS_PER_TILE)])
    plsc.subcore_barrier()

    def body(k, carry):
        pltpu.sync_copy(ones_v.at[pl.ds(0, CHUNK)], deg_sh.at[didx.at[k]],
                        add=True)
        return carry

    lax.fori_loop(0, DEG_CHUNKS, body, 0)
    plsc.subcore_barrier()

    pltpu.sync_copy(
        deg_sh.at[pl.ds(s * ROWS_PER_TILE, ROWS_PER_TILE)],
        deg_out.at[c, pl.ds(s * ROWS_PER_TILE, ROWS_PER_TILE)],
    )


def _scatter_body(scaled_hbm, sidx_hbm, didx_hbm, zeros_hbm, agg_out,
                  sidx, didx, g0, agg_sh, gs0):
    c = lax.axis_index("c")
    s = lax.axis_index("s")
    base = s * ROWS_PER_TILE
    pltpu.sync_copy(zeros_hbm, agg_sh.at[pl.ds(base, ROWS_PER_TILE)])
    plsc.subcore_barrier()

    pltpu.sync_copy(sidx_hbm.at[c, s], sidx)
    pltpu.sync_copy(didx_hbm.at[s], didx)

    def body(k, carry):
        pltpu.async_copy(scaled_hbm.at[sidx.at[k]], g0, gs0).wait()
        pltpu.sync_copy(g0, agg_sh.at[didx.at[k]], add=True)
        return carry

    lax.fori_loop(0, N_CHUNKS, body, 0)

    plsc.subcore_barrier()

    pltpu.sync_copy(
        agg_sh.at[pl.ds(base, ROWS_PER_TILE)],
        agg_out.at[c, pl.ds(base, ROWS_PER_TILE)],
    )


@functools.lru_cache(maxsize=1)
def _sc_kernels():
    mesh = plsc.VectorSubcoreMesh(core_axis_name="c", subcore_axis_name="s")
    deg_kernel = pl.kernel(
        _deg_body,
        out_type=jax.ShapeDtypeStruct((2, N_PAD), jnp.float32),
        mesh=mesh,
        scratch_types=[
            pltpu.VMEM((DEG_CHUNKS, CHUNK), jnp.int32),
            pltpu.VMEM((128,), jnp.float32),
            pltpu.VMEM((ROWS_PER_TILE,), jnp.float32),
            pltpu.VMEM_SHARED((N_PAD,), jnp.float32),
        ],
    )
    scatter_kernel = pl.kernel(
        _scatter_body,
        out_type=jax.ShapeDtypeStruct((2, N_PAD, H), jnp.float32),
        mesh=mesh,
        scratch_types=[
            pltpu.VMEM((N_CHUNKS, CHUNK), jnp.int32),
            pltpu.VMEM((N_CHUNKS, CHUNK), jnp.int32),
            pltpu.VMEM((CHUNK, H), jnp.float32),
            pltpu.VMEM_SHARED((N_PAD, H), jnp.float32),
            pltpu.SemaphoreType.DMA,
        ],
    )
    return deg_kernel, scatter_kernel


def _l1_body(nt_ref, ch_ref, T_ref, deg_ref, W_ref, b_ref, out_ref, dinv_ref):
    nt = nt_ref[...]
    ch = ch_ref[...]
    ids = lax.broadcasted_iota(jnp.int32, (BN_ROWS, D), 1)
    m = (ids == nt).astype(jnp.float32) + (ids == (ch + NT)).astype(jnp.float32)
    x = jnp.dot(m, T_ref[...], preferred_element_type=jnp.float32)
    deg = deg_ref[0, :] + deg_ref[1, :] + 1.0
    dinv = lax.rsqrt(deg)[:, None]
    y = (jnp.dot(x, W_ref[...], preferred_element_type=jnp.float32)
         + b_ref[...]) * dinv
    out_ref[0] = y[:, :H]
    out_ref[1] = y[:, H:]
    dinv_ref[...] = dinv


def _li_body(agg_ref, scp_ref, dinv_ref, g_ref, be_ref, mu_ref, va_ref,
             W_ref, b_ref, out_ref):
    dinv = dinv_ref[...]
    hpre = jnp.concatenate(
        [agg_ref[0] + scp_ref[0], agg_ref[1] + scp_ref[1]], axis=1) * dinv
    sbn = g_ref[...] * lax.rsqrt(va_ref[...] + EPS)
    h = sbn * (hpre - mu_ref[...]) + be_ref[...]
    h = jnp.maximum(h, 0.0)
    y = (jnp.dot(h, W_ref[...], preferred_element_type=jnp.float32)
         + b_ref[...]) * dinv
    out_ref[0] = y[:, :H]
    out_ref[1] = y[:, H:]


def _ep_body(agg_ref, scp_ref, dinv_ref, g_ref, be_ref, mu_ref, va_ref,
             out_ref):
    dinv = dinv_ref[...]
    hpre = jnp.concatenate(
        [agg_ref[0] + scp_ref[0], agg_ref[1] + scp_ref[1]], axis=1) * dinv
    sbn = g_ref[...] * lax.rsqrt(va_ref[...] + EPS)
    out_ref[...] = sbn * (hpre - mu_ref[...]) + be_ref[...]


def _col_spec(i):
    return (i, 0)


_spec_rows1 = pl.BlockSpec((BN_ROWS, 1), lambda i: (i, 0))
_spec_deg = pl.BlockSpec((2, BN_ROWS), lambda i: (0, i))
_spec_mat = pl.BlockSpec((D, D), lambda i: (0, 0))
_spec_vec = pl.BlockSpec((1, D), lambda i: (0, 0))
_spec_split = pl.BlockSpec((2, BN_ROWS, H), lambda i: (0, i, 0))
_spec_full = pl.BlockSpec((BN_ROWS, D), lambda i: (i, 0))

_l1_call = pl.pallas_call(
    _l1_body,
    grid=(N_BLOCKS,),
    in_specs=[_spec_rows1, _spec_rows1, _spec_mat, _spec_deg, _spec_mat,
              _spec_vec],
    out_specs=[_spec_split, _spec_rows1],
    out_shape=[
        jax.ShapeDtypeStruct((2, N_PAD, H), jnp.float32),
        jax.ShapeDtypeStruct((N_PAD, 1), jnp.float32),
    ],
)

_li_call = pl.pallas_call(
    _li_body,
    grid=(N_BLOCKS,),
    in_specs=[_spec_split, _spec_split, _spec_rows1, _spec_vec, _spec_vec,
              _spec_vec, _spec_vec, _spec_mat, _spec_vec],
    out_specs=_spec_split,
    out_shape=jax.ShapeDtypeStruct((2, N_PAD, H), jnp.float32),
)

_ep_call = pl.pallas_call(
    _ep_body,
    grid=(N_BLOCKS,),
    in_specs=[_spec_split, _spec_split, _spec_rows1, _spec_vec, _spec_vec,
              _spec_vec, _spec_vec],
    out_specs=_spec_full,
    out_shape=jax.ShapeDtypeStruct((N_PAD, D), jnp.float32),
)


@jax.jit
def kernel(node_types, chirality, edge_index, node_emb, chir_emb, W, b,
           gamma, beta, run_mean, run_var):
    nt = jnp.zeros((N_PAD, 1), jnp.int32).at[:N, 0].set(
        node_types.astype(jnp.int32))
    ch = jnp.zeros((N_PAD, 1), jnp.int32).at[:N, 0].set(
        chirality.astype(jnp.int32))
    src = edge_index[0].astype(jnp.int32)
    dst = edge_index[1].astype(jnp.int32)

    T = jnp.zeros((D, D), jnp.float32)
    T = T.at[:NT].set(node_emb).at[NT:NT + NC].set(chir_emb)

    dst_deg = dst.reshape(2, 16, DEG_CHUNKS, CHUNK)

    srcp = src.reshape(16, N_CHUNKS, CHUNK)
    sidx = jnp.stack([srcp, srcp + N_PAD])
    didx = dst.reshape(16, N_CHUNKS, CHUNK)

    zeros_blk = jnp.zeros((ROWS_PER_TILE, H), jnp.float32)

    deg_kernel, scatter_kernel = _sc_kernels()
    deg2 = deg_kernel(dst_deg)

    scaled, dinv = _l1_call(nt, ch, T, deg2, W[0], b[0][None, :])
    for i in range(1, L + 1):
        agg = scatter_kernel(scaled.reshape(2 * N_PAD, H), sidx, didx,
                             zeros_blk)
        if i < L:
            scaled = _li_call(agg, scaled, dinv,
                              gamma[i - 1][None, :], beta[i - 1][None, :],
                              run_mean[i - 1][None, :], run_var[i - 1][None, :],
                              W[i], b[i][None, :])
        else:
            out = _ep_call(agg, scaled, dinv,
                           gamma[L - 1][None, :], beta[L - 1][None, :],
                           run_mean[L - 1][None, :], run_var[L - 1][None, :])
    return out[:N]


# speedup vs baseline: 6.7588x; 3.8364x over previous
import functools

import jax
import jax.numpy as jnp
from jax import lax
from jax.experimental import pallas as pl
from jax.experimental.pallas import tpu as pltpu
from jax.experimental.pallas import tpu_sc as plsc

N = 10000
E = 160000
D = 256
H = 128
L = 5
NT = 120
NC = 4
EPS = 1e-5

N_PAD = 10240
BN_ROWS = 512
N_BLOCKS = N_PAD // BN_ROWS
ROWS_PER_TILE = N_PAD // 16

CHUNK = 125
N_CHUNKS = 80
E_TILE = N_CHUNKS * CHUNK

DEG_CHUNKS = 40

def _deg_body(dst_hbm, deg_out, didx, ones_v, zbuf, deg_sh):
    c = lax.axis_index("c")
    s = lax.axis_index("s")
    pltpu.sync_copy(dst_hbm.at[c, s], didx)

    z16 = jnp.zeros((16,), jnp.float32)
    o16 = jnp.ones((16,), jnp.float32)

    def fill_ones(j, carry):
        ones_v[pl.ds(j * 16, 16)] = o16
        return carry

    lax.fori_loop(0, 8, fill_ones, 0)

    def fill_zero(j, carry):
        zbuf[pl.ds(j * 16, 16)] = z16
        return carry

    lax.fori_loop(0, ROWS_PER_TILE // 16, fill_zero, 0)

    pltpu.sync_copy(zbuf, deg_sh.at[pl.ds(s * ROWS_PER_TILE, ROWS_PER_TILE)])
    plsc.subcore_barrier()

    def body(k, carry):
        pltpu.sync_copy(ones_v.at[pl.ds(0, CHUNK)], deg_sh.at[didx.at[k]],
                        add=True)
        return carry

    lax.fori_loop(0, DEG_CHUNKS, body, 0)
    plsc.subcore_barrier()

    pltpu.sync_copy(
        deg_sh.at[pl.ds(s * ROWS_PER_TILE, ROWS_PER_TILE)],
        deg_out.at[c, pl.ds(s * ROWS_PER_TILE, ROWS_PER_TILE)],
    )


def _scatter_body(scaled_hbm, sidx_hbm, didx_hbm, zeros_hbm, agg_out,
                  sidx, didx, g0, agg_sh, gs0):
    c = lax.axis_index("c")
    s = lax.axis_index("s")
    base = s * ROWS_PER_TILE
    pltpu.sync_copy(zeros_hbm, agg_sh.at[pl.ds(base, ROWS_PER_TILE)])
    plsc.subcore_barrier()

    for h in range(2):
        pltpu.sync_copy(sidx_hbm.at[c, s, h], sidx)
        pltpu.sync_copy(didx_hbm.at[s, h], didx)

        @functools.partial(plsc.parallel_loop, 0, N_CHUNKS // 2, unroll=2)
        def _loop(k):
            half = lax.rem(k, 2)
            pltpu.async_copy(
                scaled_hbm.at[sidx.at[k]], g0.at[half], gs0).wait()
            pltpu.sync_copy(g0.at[half], agg_sh.at[didx.at[k]], add=True)

    plsc.subcore_barrier()

    pltpu.sync_copy(
        agg_sh.at[pl.ds(base, ROWS_PER_TILE)],
        agg_out.at[c, pl.ds(base, ROWS_PER_TILE)],
    )


@functools.lru_cache(maxsize=1)
def _sc_kernels():
    mesh = plsc.VectorSubcoreMesh(core_axis_name="c", subcore_axis_name="s")
    deg_kernel = pl.kernel(
        _deg_body,
        out_type=jax.ShapeDtypeStruct((2, N_PAD), jnp.float32),
        mesh=mesh,
        scratch_types=[
            pltpu.VMEM((DEG_CHUNKS, CHUNK), jnp.int32),
            pltpu.VMEM((128,), jnp.float32),
            pltpu.VMEM((ROWS_PER_TILE,), jnp.float32),
            pltpu.VMEM_SHARED((N_PAD,), jnp.float32),
        ],
    )
    scatter_kernel = pl.kernel(
        _scatter_body,
        out_type=jax.ShapeDtypeStruct((2, N_PAD, H), jnp.float32),
        mesh=mesh,
        scratch_types=[
            pltpu.VMEM((N_CHUNKS // 2, CHUNK), jnp.int32),
            pltpu.VMEM((N_CHUNKS // 2, CHUNK), jnp.int32),
            pltpu.VMEM((2, CHUNK, H), jnp.float32),
            pltpu.VMEM_SHARED((N_PAD, H), jnp.float32),
            pltpu.SemaphoreType.DMA,
        ],
    )
    return deg_kernel, scatter_kernel


def _l1_body(nt_ref, ch_ref, T_ref, deg_ref, W_ref, b_ref, out_ref, dinv_ref):
    nt = nt_ref[...]
    ch = ch_ref[...]
    ids = lax.broadcasted_iota(jnp.int32, (BN_ROWS, D), 1)
    m = (ids == nt).astype(jnp.float32) + (ids == (ch + NT)).astype(jnp.float32)
    x = jnp.dot(m, T_ref[...], preferred_element_type=jnp.float32)
    deg = deg_ref[0, :] + deg_ref[1, :] + 1.0
    dinv = lax.rsqrt(deg)[:, None]
    y = (jnp.dot(x, W_ref[...], preferred_element_type=jnp.float32)
         + b_ref[...]) * dinv
    out_ref[0] = y[:, :H]
    out_ref[1] = y[:, H:]
    dinv_ref[...] = dinv


def _li_body(agg_ref, scp_ref, dinv_ref, g_ref, be_ref, mu_ref, va_ref,
             W_ref, b_ref, out_ref):
    dinv = dinv_ref[...]
    hpre = jnp.concatenate(
        [agg_ref[0] + scp_ref[0], agg_ref[1] + scp_ref[1]], axis=1) * dinv
    sbn = g_ref[...] * lax.rsqrt(va_ref[...] + EPS)
    h = sbn * (hpre - mu_ref[...]) + be_ref[...]
    h = jnp.maximum(h, 0.0)
    y = (jnp.dot(h, W_ref[...], preferred_element_type=jnp.float32)
         + b_ref[...]) * dinv
    out_ref[0] = y[:, :H]
    out_ref[1] = y[:, H:]


def _ep_body(agg_ref, scp_ref, dinv_ref, g_ref, be_ref, mu_ref, va_ref,
             out_ref):
    dinv = dinv_ref[...]
    hpre = jnp.concatenate(
        [agg_ref[0] + scp_ref[0], agg_ref[1] + scp_ref[1]], axis=1) * dinv
    sbn = g_ref[...] * lax.rsqrt(va_ref[...] + EPS)
    out_ref[...] = sbn * (hpre - mu_ref[...]) + be_ref[...]


def _col_spec(i):
    return (i, 0)


_spec_rows1 = pl.BlockSpec((BN_ROWS, 1), lambda i: (i, 0))
_spec_deg = pl.BlockSpec((2, BN_ROWS), lambda i: (0, i))
_spec_mat = pl.BlockSpec((D, D), lambda i: (0, 0))
_spec_vec = pl.BlockSpec((1, D), lambda i: (0, 0))
_spec_split = pl.BlockSpec((2, BN_ROWS, H), lambda i: (0, i, 0))
_spec_full = pl.BlockSpec((BN_ROWS, D), lambda i: (i, 0))

_l1_call = pl.pallas_call(
    _l1_body,
    grid=(N_BLOCKS,),
    in_specs=[_spec_rows1, _spec_rows1, _spec_mat, _spec_deg, _spec_mat,
              _spec_vec],
    out_specs=[_spec_split, _spec_rows1],
    out_shape=[
        jax.ShapeDtypeStruct((2, N_PAD, H), jnp.float32),
        jax.ShapeDtypeStruct((N_PAD, 1), jnp.float32),
    ],
)

_li_call = pl.pallas_call(
    _li_body,
    grid=(N_BLOCKS,),
    in_specs=[_spec_split, _spec_split, _spec_rows1, _spec_vec, _spec_vec,
              _spec_vec, _spec_vec, _spec_mat, _spec_vec],
    out_specs=_spec_split,
    out_shape=jax.ShapeDtypeStruct((2, N_PAD, H), jnp.float32),
)

_ep_call = pl.pallas_call(
    _ep_body,
    grid=(N_BLOCKS,),
    in_specs=[_spec_split, _spec_split, _spec_rows1, _spec_vec, _spec_vec,
              _spec_vec, _spec_vec],
    out_specs=_spec_full,
    out_shape=jax.ShapeDtypeStruct((N_PAD, D), jnp.float32),
)


@jax.jit
def kernel(node_types, chirality, edge_index, node_emb, chir_emb, W, b,
           gamma, beta, run_mean, run_var):
    nt = jnp.zeros((N_PAD, 1), jnp.int32).at[:N, 0].set(
        node_types.astype(jnp.int32))
    ch = jnp.zeros((N_PAD, 1), jnp.int32).at[:N, 0].set(
        chirality.astype(jnp.int32))
    src = edge_index[0].astype(jnp.int32)
    dst = edge_index[1].astype(jnp.int32)

    T = jnp.zeros((D, D), jnp.float32)
    T = T.at[:NT].set(node_emb).at[NT:NT + NC].set(chir_emb)

    dst_deg = dst.reshape(2, 16, DEG_CHUNKS, CHUNK)

    srcp = src.reshape(16, 2, N_CHUNKS // 2, CHUNK)
    sidx = jnp.stack([srcp, srcp + N_PAD])
    didx = dst.reshape(16, 2, N_CHUNKS // 2, CHUNK)

    zeros_blk = jnp.zeros((ROWS_PER_TILE, H), jnp.float32)

    deg_kernel, scatter_kernel = _sc_kernels()
    deg2 = deg_kernel(dst_deg)

    scaled, dinv = _l1_call(nt, ch, T, deg2, W[0], b[0][None, :])
    for i in range(1, L + 1):
        agg = scatter_kernel(scaled.reshape(2 * N_PAD, H), sidx, didx,
                             zeros_blk)
        if i < L:
            scaled = _li_call(agg, scaled, dinv,
                              gamma[i - 1][None, :], beta[i - 1][None, :],
                              run_mean[i - 1][None, :], run_var[i - 1][None, :],
                              W[i], b[i][None, :])
        else:
            out = _ep_call(agg, scaled, dinv,
                           gamma[L - 1][None, :], beta[L - 1][None, :],
                           run_mean[L - 1][None, :], run_var[L - 1][None, :])
    return out[:N]
